# baseline (speedup 1.0000x reference)
import functools

import jax
import jax.numpy as jnp
from jax import lax
from jax.experimental import pallas as pl
from jax.experimental.pallas import tpu as pltpu
from jax.experimental.pallas import tpu_sc as plsc

N_NODES = 10000
N_EDGES = 160000
D = 256
DH = D // 2
EPS = 1e-5

LANES = 16
E_BLK = 80
N_CHUNKS = N_EDGES // E_BLK
NW = 32
MAX_STEPS = -(-N_CHUNKS // NW)


def _ln_body(x_ref, g_ref, b_ref, o_ref):
    x = x_ref[...]
    mean = jnp.mean(x, axis=-1, keepdims=True)
    var = jnp.mean((x - mean) ** 2, axis=-1, keepdims=True)
    xn = (x - mean) * lax.rsqrt(var + EPS) * g_ref[...] + b_ref[...]
    lo = lax.bitcast_convert_type(xn[:, :DH].astype(jnp.bfloat16), jnp.uint16)
    hi = lax.bitcast_convert_type(xn[:, DH:].astype(jnp.bfloat16), jnp.uint16)
    packed = lo.astype(jnp.uint32) | (hi.astype(jnp.uint32) << 16)
    o_ref[...] = lax.bitcast_convert_type(packed, jnp.int32)


def _layernorm_packed(x, gamma, beta):
    blk = 2000
    return pl.pallas_call(
        _ln_body,
        grid=(N_NODES // blk,),
        in_specs=[
            pl.BlockSpec((blk, D), lambda i: (i, 0)),
            pl.BlockSpec((D,), lambda i: (0,)),
            pl.BlockSpec((D,), lambda i: (0,)),
        ],
        out_specs=pl.BlockSpec((blk, DH), lambda i: (i, 0)),
        out_shape=jax.ShapeDtypeStruct((N_NODES, DH), jnp.int32),
    )(x, gamma, beta)


_MESH = plsc.VectorSubcoreMesh(core_axis_name="c", subcore_axis_name="s")

_HI_MASK = -65536


@functools.partial(
    pl.kernel,
    out_type=jax.ShapeDtypeStruct((N_EDGES, D), jnp.float32),
    mesh=_MESH,
    scratch_types=[
        pltpu.VMEM((E_BLK,), jnp.int32),
        pltpu.VMEM((E_BLK,), jnp.int32),
        pltpu.VMEM((E_BLK, DH), jnp.int32),
        pltpu.VMEM((E_BLK, DH), jnp.int32),
        pltpu.VMEM((E_BLK, D), jnp.float32),
        pltpu.VMEM((E_BLK,), jnp.int32),
        pltpu.VMEM((E_BLK,), jnp.int32),
        pltpu.VMEM((E_BLK, DH), jnp.int32),
        pltpu.VMEM((E_BLK, DH), jnp.int32),
        pltpu.VMEM((E_BLK, D), jnp.float32),
        pltpu.SemaphoreType.DMA,
        pltpu.SemaphoreType.DMA,
        pltpu.SemaphoreType.DMA,
        pltpu.SemaphoreType.DMA,
        pltpu.SemaphoreType.DMA,
        pltpu.SemaphoreType.DMA,
    ],
)
def _gather_mul(xn_hbm, src_hbm, dst_hbm, out_hbm,
                si0, di0, a0, b0, o0, si1, di1, a1, b1, o1,
                gi0, gi1, g0, g1, w0, w1):
    wid = lax.axis_index("s") * 2 + lax.axis_index("c")
    SI, DI, A, B, O = (si0, si1), (di0, di1), (a0, a1), (b0, b1), (o0, o1)
    GI, G, W = (gi0, gi1), (g0, g1), (w0, w1)

    def cbase(s):
        return (wid + s * NW) * E_BLK

    def valid(s):
        return (wid + s * NW) < N_CHUNKS

    def fire_idx(s, k):
        pltpu.async_copy(src_hbm.at[pl.ds(cbase(s), E_BLK)], SI[k], GI[k])
        pltpu.async_copy(dst_hbm.at[pl.ds(cbase(s), E_BLK)], DI[k], GI[k])

    def wait_idx(k):
        pltpu.make_async_copy(src_hbm.at[pl.ds(0, E_BLK)], SI[k], GI[k]).wait()
        pltpu.make_async_copy(dst_hbm.at[pl.ds(0, E_BLK)], DI[k], GI[k]).wait()

    def fire_gathers(k):
        pltpu.async_copy(xn_hbm.at[SI[k]], A[k], G[k])
        pltpu.async_copy(xn_hbm.at[DI[k]], B[k], G[k])

    def wait_gathers(k):
        pltpu.make_async_copy(xn_hbm.at[pl.ds(0, E_BLK)], A[k], G[k]).wait()
        pltpu.make_async_copy(xn_hbm.at[pl.ds(0, E_BLK)], B[k], G[k]).wait()

    def fire_wb(s, k):
        pltpu.async_copy(O[k], out_hbm.at[pl.ds(cbase(s), E_BLK)], W[k])

    def wait_wb(k):
        pltpu.make_async_copy(O[k], out_hbm.at[pl.ds(0, E_BLK)], W[k]).wait()

    def multiply(k):
        ak, bk, ok = A[k], B[k], O[k]

        @plsc.parallel_loop(0, E_BLK, unroll=4)
        def _row(e):
            for j in range(DH // LANES):
                sl = pl.ds(j * LANES, LANES)
                wa = ak[e, sl]
                wb = bk[e, sl]
                a_lo = lax.bitcast_convert_type(wa << 16, jnp.float32)
                b_lo = lax.bitcast_convert_type(wb << 16, jnp.float32)
                a_hi = lax.bitcast_convert_type(wa & _HI_MASK, jnp.float32)
                b_hi = lax.bitcast_convert_type(wb & _HI_MASK, jnp.float32)
                ok[e, sl] = a_lo * b_lo
                ok[e, pl.ds(DH + j * LANES, LANES)] = a_hi * b_hi

    fire_idx(0, 0)
    fire_idx(1, 1)
    wait_idx(0)
    fire_gathers(0)

    def step(s, k):
        kn = 1 - k

        @pl.when(jnp.logical_and(s >= 1, valid(s - 1)))
        def _():
            wait_wb(kn)

        @pl.when(valid(s + 1))
        def _():
            wait_idx(kn)
            fire_gathers(kn)

        @pl.when(valid(s))
        def _():
            wait_gathers(k)

        @pl.when(valid(s + 2))
        def _():
            fire_idx(s + 2, k)

        @pl.when(valid(s))
        def _():
            multiply(k)
            fire_wb(s, k)

    def pair(i, _):
        step(2 * i, 0)
        step(2 * i + 1, 1)
        return 0

    lax.fori_loop(0, (MAX_STEPS + 2) // 2, pair, 0)


def kernel(data, x, edge, gamma, beta):
    xn_packed = _layernorm_packed(x, gamma, beta)
    src = edge[0]
    dst = edge[1]
    return _gather_mul(xn_packed, src, dst)


# speedup vs baseline: 1.0378x; 1.0378x over previous
import functools

import jax
import jax.numpy as jnp
from jax import lax
from jax.experimental import pallas as pl
from jax.experimental.pallas import tpu as pltpu
from jax.experimental.pallas import tpu_sc as plsc

N_NODES = 10000
N_EDGES = 160000
D = 256
DH = D // 2
EPS = 1e-5

LANES = 16
E_BLK = 80
N_CHUNKS = N_EDGES // E_BLK
NW = 32
MAX_STEPS = -(-N_CHUNKS // NW)
NB = 3

_HI_MASK = -65536


def _ln_body(x_ref, g_ref, b_ref, o_ref):
    x = x_ref[...]
    mean = jnp.mean(x, axis=-1, keepdims=True)
    var = jnp.mean((x - mean) ** 2, axis=-1, keepdims=True)
    xn = (x - mean) * lax.rsqrt(var + EPS) * g_ref[...] + b_ref[...]
    lo = lax.bitcast_convert_type(xn[:, :DH].astype(jnp.bfloat16), jnp.uint16)
    hi = lax.bitcast_convert_type(xn[:, DH:].astype(jnp.bfloat16), jnp.uint16)
    packed = lo.astype(jnp.uint32) | (hi.astype(jnp.uint32) << 16)
    o_ref[...] = lax.bitcast_convert_type(packed, jnp.int32)


def _layernorm_packed(x, gamma, beta):
    blk = 2000
    return pl.pallas_call(
        _ln_body,
        grid=(N_NODES // blk,),
        in_specs=[
            pl.BlockSpec((blk, D), lambda i: (i, 0)),
            pl.BlockSpec((D,), lambda i: (0,)),
            pl.BlockSpec((D,), lambda i: (0,)),
        ],
        out_specs=pl.BlockSpec((blk, DH), lambda i: (i, 0)),
        out_shape=jax.ShapeDtypeStruct((N_NODES, DH), jnp.int32),
    )(x, gamma, beta)


_MESH = plsc.VectorSubcoreMesh(core_axis_name="c", subcore_axis_name="s")


@functools.partial(
    pl.kernel,
    out_type=jax.ShapeDtypeStruct((N_EDGES, D), jnp.float32),
    mesh=_MESH,
    scratch_types=(
        [pltpu.VMEM((E_BLK,), jnp.int32)] * (2 * NB)
        + [pltpu.VMEM((E_BLK, DH), jnp.int32)] * (2 * NB)
        + [pltpu.VMEM((E_BLK, D), jnp.float32)] * NB
        + [pltpu.SemaphoreType.DMA] * (3 * NB)
    ),
)
def _gather_mul(xn_hbm, src_hbm, dst_hbm, out_hbm,
                si0, di0, si1, di1, si2, di2,
                a0, b0, a1, b1, a2, b2, o0, o1, o2,
                gi0, gi1, gi2, g0, g1, g2, w0, w1, w2):
    wid = lax.axis_index("s") * 2 + lax.axis_index("c")
    SI, DI = (si0, si1, si2), (di0, di1, di2)
    A, B, O = (a0, a1, a2), (b0, b1, b2), (o0, o1, o2)
    GI, G, W = (gi0, gi1, gi2), (g0, g1, g2), (w0, w1, w2)

    def cbase(s):
        return (wid + s * NW) * E_BLK

    def valid(s):
        return (wid + s * NW) < N_CHUNKS

    def fire_idx(s, k):
        pltpu.async_copy(src_hbm.at[pl.ds(cbase(s), E_BLK)], SI[k], GI[k])
        pltpu.async_copy(dst_hbm.at[pl.ds(cbase(s), E_BLK)], DI[k], GI[k])

    def wait_idx(k):
        pltpu.make_async_copy(src_hbm.at[pl.ds(0, E_BLK)], SI[k], GI[k]).wait()
        pltpu.make_async_copy(dst_hbm.at[pl.ds(0, E_BLK)], DI[k], GI[k]).wait()

    def fire_gathers(k):
        pltpu.async_copy(xn_hbm.at[SI[k]], A[k], G[k])
        pltpu.async_copy(xn_hbm.at[DI[k]], B[k], G[k])

    def wait_gathers(k):
        pltpu.make_async_copy(xn_hbm.at[pl.ds(0, E_BLK)], A[k], G[k]).wait()
        pltpu.make_async_copy(xn_hbm.at[pl.ds(0, E_BLK)], B[k], G[k]).wait()

    def fire_wb(s, k):
        pltpu.async_copy(O[k], out_hbm.at[pl.ds(cbase(s), E_BLK)], W[k])

    def wait_wb(k):
        pltpu.make_async_copy(O[k], out_hbm.at[pl.ds(0, E_BLK)], W[k]).wait()

    def multiply(k):
        ak, bk, ok = A[k], B[k], O[k]

        @plsc.parallel_loop(0, E_BLK, unroll=4)
        def _row(e):
            for j in range(DH // LANES):
                sl = pl.ds(j * LANES, LANES)
                wa = ak[e, sl]
                wb = bk[e, sl]
                a_lo = lax.bitcast_convert_type(wa << 16, jnp.float32)
                b_lo = lax.bitcast_convert_type(wb << 16, jnp.float32)
                a_hi = lax.bitcast_convert_type(wa & _HI_MASK, jnp.float32)
                b_hi = lax.bitcast_convert_type(wb & _HI_MASK, jnp.float32)
                ok[e, sl] = a_lo * b_lo
                ok[e, pl.ds(DH + j * LANES, LANES)] = a_hi * b_hi

    fire_idx(0, 0)
    fire_idx(1, 1)
    fire_idx(2, 2)
    wait_idx(0)
    fire_gathers(0)
    wait_idx(1)
    fire_gathers(1)

    def step(s, k):
        k2 = (k + 2) % NB

        @pl.when(valid(s + 2))
        def _():
            wait_idx(k2)
            fire_gathers(k2)

        @pl.when(valid(s))
        def _():
            wait_gathers(k)

        @pl.when(valid(s + 3))
        def _():
            fire_idx(s + 3, k)

        @pl.when(jnp.logical_and(s >= 3, valid(s - 3)))
        def _():
            wait_wb(k)

        @pl.when(valid(s))
        def _():
            multiply(k)
            fire_wb(s, k)

    def triple(i, _):
        s0 = 3 * i
        step(s0, 0)
        step(s0 + 1, 1)
        step(s0 + 2, 2)
        return 0

    lax.fori_loop(0, (MAX_STEPS + 3 + 2) // 3, triple, 0)


def kernel(data, x, edge, gamma, beta):
    xn_packed = _layernorm_packed(x, gamma, beta)
    src = edge[0]
    dst = edge[1]
    return _gather_mul(xn_packed, src, dst)
